# Initial kernel scaffold; baseline (speedup 1.0000x reference)
import functools

import jax
import jax.numpy as jnp
from jax import lax
from jax.experimental import pallas as pl
from jax.experimental.pallas import tpu as pltpu
from jax.experimental.pallas import tpu_sc as plsc

N = 10000
D = 128
E = 320000
G = 64
C = 10

NC, NS = 2, 16
NW = NC * NS
JG = 80
EPT = JG * 128
E_PAD = NW * EPT
NPAD = 10016
ZPT = NPAD // NS
OPT = 625


def _agg_body(h_hbm, src_hbm, dst_hbm, zero_hbm, out0, out1,
              src_v, dst_v, rows_v, acc_sh, sem0, sem1):
    cid = lax.axis_index("c")
    sid = lax.axis_index("s")
    wid = cid * NS + sid

    zbase = sid * ZPT
    for g in range(5):
        nr = 128 if g < 4 else ZPT - 4 * 128
        sl = pl.ds(zbase + g * 128, nr)
        pltpu.sync_copy(zero_hbm.at[sl], rows_v.at[0, pl.ds(0, nr)])
        pltpu.sync_copy(rows_v.at[0, pl.ds(0, nr)], acc_sh.at[sl])
    plsc.subcore_barrier()

    pltpu.sync_copy(src_hbm.at[wid], src_v)
    pltpu.sync_copy(dst_hbm.at[wid], dst_v)

    sems = (sem0, sem1)

    def gather(j, b):
        return pltpu.make_async_copy(h_hbm.at[src_v.at[j]], rows_v.at[b], sems[b])

    gather(0, 0).start()

    def body(i, carry):
        j0 = 2 * i
        gather(j0, 0).wait()
        gather(j0 + 1, 1).start()
        pltpu.sync_copy(rows_v.at[0], acc_sh.at[dst_v.at[j0]], add=True)
        gather(j0 + 1, 1).wait()

        @pl.when(i < JG // 2 - 1)
        def _():
            gather(j0 + 2, 0).start()

        pltpu.sync_copy(rows_v.at[1], acc_sh.at[dst_v.at[j0 + 1]], add=True)
        return carry

    lax.fori_loop(0, JG // 2, body, 0)
    plsc.subcore_barrier()

    obase = sid * OPT
    for g in range(5):
        sl = pl.ds(obase + g * 125, 125)

        @pl.when(cid == 0)
        def _():
            pltpu.sync_copy(acc_sh.at[sl], rows_v.at[0, pl.ds(0, 125)])
            pltpu.sync_copy(rows_v.at[0, pl.ds(0, 125)], out0.at[sl])

        @pl.when(cid == 1)
        def _():
            pltpu.sync_copy(acc_sh.at[sl], rows_v.at[1, pl.ds(0, 125)])
            pltpu.sync_copy(rows_v.at[1, pl.ds(0, 125)], out1.at[sl])


_agg = functools.partial(
    pl.kernel,
    out_type=(jax.ShapeDtypeStruct((N, D), jnp.float32),
              jax.ShapeDtypeStruct((N, D), jnp.float32)),
    mesh=plsc.VectorSubcoreMesh(core_axis_name="c", subcore_axis_name="s"),
    scratch_types=[
        pltpu.VMEM((JG, 128), jnp.int32),
        pltpu.VMEM((JG, 128), jnp.int32),
        pltpu.VMEM((2, 128, D), jnp.float32),
        pltpu.VMEM_SHARED((NPAD, D), jnp.float32),
        pltpu.SemaphoreType.DMA,
        pltpu.SemaphoreType.DMA,
    ],
)(_agg_body)


R = 1000


def _mlp_body(x_ref, a0_ref, a1_ref, w1_ref, b1_ref, w2_ref, b2_ref, o_ref):
    z = x_ref[...] + a0_ref[...] + a1_ref[...]
    t = jnp.dot(z, w1_ref[...], preferred_element_type=jnp.float32) + b1_ref[...]
    t = jnp.maximum(t, 0.0)
    h = jnp.dot(t, w2_ref[...], preferred_element_type=jnp.float32) + b2_ref[...]
    o_ref[...] = jnp.maximum(h, 0.0)


def _mlp(x, a0, a1, W1, b1, W2, b2, *, interpret=False):
    return pl.pallas_call(
        _mlp_body,
        grid=(N // R,),
        in_specs=[pl.BlockSpec((R, D), lambda i: (i, 0))] * 3 + [
            pl.BlockSpec((D, D), lambda i: (0, 0)),
            pl.BlockSpec((1, D), lambda i: (0, 0)),
            pl.BlockSpec((D, D), lambda i: (0, 0)),
            pl.BlockSpec((1, D), lambda i: (0, 0)),
        ],
        out_specs=pl.BlockSpec((R, D), lambda i: (i, 0)),
        out_shape=jax.ShapeDtypeStruct((N, D), jnp.float32),
        interpret=interpret,
    )(x, a0, a1, W1, b1.reshape(1, D), W2, b2.reshape(1, D))


def _pool_body(h_ref, batch_ref, wl_ref, bl_ref, o_ref):
    h = h_ref[...]
    b = batch_ref[...]
    gids = lax.broadcasted_iota(jnp.int32, (G, N), 0)
    mask = (gids == b).astype(jnp.float32)
    sums = jnp.dot(mask, h, preferred_element_type=jnp.float32)
    counts = jnp.sum(mask, axis=1, keepdims=True)
    mean = sums / jnp.maximum(counts, 1.0)
    p = jnp.dot(mean, wl_ref[...], preferred_element_type=jnp.float32) + bl_ref[...]
    m = jnp.max(p, axis=1, keepdims=True)
    lse = m + jnp.log(jnp.sum(jnp.exp(p - m), axis=1, keepdims=True))
    o_ref[...] = p - lse


def _pool(h, batch2d, Wl, bl, *, interpret=False):
    return pl.pallas_call(
        _pool_body,
        in_specs=[
            pl.BlockSpec((N, D), lambda: (0, 0)),
            pl.BlockSpec((1, N), lambda: (0, 0)),
            pl.BlockSpec((D, C), lambda: (0, 0)),
            pl.BlockSpec((1, C), lambda: (0, 0)),
        ],
        out_specs=pl.BlockSpec((G, C), lambda: (0, 0)),
        out_shape=jax.ShapeDtypeStruct((G, C), jnp.float32),
        interpret=interpret,
    )(h, batch2d, Wl, bl.reshape(1, C))


def kernel(x, edge_index, batch, W1a, b1a, W2a, b2a, W1b, b1b, W2b, b2b, Wl, bl):
    src = edge_index[0]
    dst = edge_index[1]
    pad = E_PAD - E
    srcp = jnp.concatenate([src, jnp.zeros((pad,), jnp.int32)]).reshape(NW, JG, 128)
    dstp = jnp.concatenate([dst, jnp.full((pad,), N, jnp.int32)]).reshape(NW, JG, 128)
    zeros = jnp.zeros((NPAD, D), jnp.float32)

    a0, a1 = _agg(x, srcp, dstp, zeros)
    h1 = _mlp(x, a0, a1, W1a, b1a, W2a, b2a)
    a0, a1 = _agg(h1, srcp, dstp, zeros)
    h2 = _mlp(h1, a0, a1, W1b, b1b, W2b, b2b)
    return _pool(h2, batch.reshape(1, N), Wl, bl)


# speedup vs baseline: 3.8420x; 3.8420x over previous
import functools

import jax
import jax.numpy as jnp
from jax import lax
from jax.experimental import pallas as pl
from jax.experimental.pallas import tpu as pltpu
from jax.experimental.pallas import tpu_sc as plsc

N = 10000
D = 128
E = 320000
G = 64
C = 10

NC, NS = 2, 16
DH = 64
JG = 160
EPT = JG * 128
E_PAD = NS * EPT
NPAD = 10240
ZPT = NPAD // NS


def _agg_body(hv_hbm, src_hbm, dst_hbm, zero_hbm, out0, out1,
              src_v, dst_v, rows_v, acc_sh, sem0, sem1):
    cid = lax.axis_index("c")
    sid = lax.axis_index("s")

    zbase = sid * ZPT
    for g in range(5):
        sl = pl.ds(zbase + g * 128, 128)
        pltpu.sync_copy(zero_hbm.at[sl], rows_v.at[0])
        pltpu.sync_copy(rows_v.at[0], acc_sh.at[sl])
    plsc.subcore_barrier()

    pltpu.sync_copy(src_hbm.at[cid, sid], src_v)
    pltpu.sync_copy(dst_hbm.at[sid], dst_v)

    sems = (sem0, sem1)

    def gather(j, b):
        return pltpu.make_async_copy(hv_hbm.at[src_v.at[j]], rows_v.at[b], sems[b])

    gather(0, 0).start()

    def body(i, carry):
        j0 = 2 * i
        gather(j0, 0).wait()
        gather(j0 + 1, 1).start()
        pltpu.sync_copy(rows_v.at[0], acc_sh.at[dst_v.at[j0]], add=True)
        gather(j0 + 1, 1).wait()

        @pl.when(i < JG // 2 - 1)
        def _():
            gather(j0 + 2, 0).start()

        pltpu.sync_copy(rows_v.at[1], acc_sh.at[dst_v.at[j0 + 1]], add=True)
        return carry

    lax.fori_loop(0, JG // 2, body, 0)
    plsc.subcore_barrier()

    for g in range(5):
        sl = pl.ds(zbase + g * 128, 128)

        @pl.when(cid == 0)
        def _():
            pltpu.sync_copy(acc_sh.at[sl], rows_v.at[0])
            pltpu.sync_copy(rows_v.at[0], out0.at[sl])

        @pl.when(cid == 1)
        def _():
            pltpu.sync_copy(acc_sh.at[sl], rows_v.at[1])
            pltpu.sync_copy(rows_v.at[1], out1.at[sl])


@functools.cache
def _make_agg():
    return functools.partial(
        pl.kernel,
        out_type=(jax.ShapeDtypeStruct((NPAD, DH), jnp.float32),
                  jax.ShapeDtypeStruct((NPAD, DH), jnp.float32)),
        mesh=plsc.VectorSubcoreMesh(core_axis_name="c", subcore_axis_name="s",
                                    num_cores=NC, num_subcores=NS),
        compiler_params=pltpu.CompilerParams(use_tc_tiling_on_sc=False),
        scratch_types=[
            pltpu.VMEM((JG, 128), jnp.int32),
            pltpu.VMEM((JG, 128), jnp.int32),
            pltpu.VMEM((2, 128, DH), jnp.float32),
            pltpu.VMEM_SHARED((NPAD, DH), jnp.float32),
            pltpu.SemaphoreType.DMA,
            pltpu.SemaphoreType.DMA,
        ],
    )(_agg_body)


def _agg(*args):
    return _make_agg()(*args)


R = 1000


def _mlp_body(x_ref, a0_ref, a1_ref, w1_ref, b1_ref, w2_ref, b2_ref, o_ref):
    z = x_ref[...] + jnp.concatenate([a0_ref[...], a1_ref[...]], axis=1)
    t = jnp.dot(z, w1_ref[...], preferred_element_type=jnp.float32) + b1_ref[...]
    t = jnp.maximum(t, 0.0)
    h = jnp.dot(t, w2_ref[...], preferred_element_type=jnp.float32) + b2_ref[...]
    o_ref[...] = jnp.maximum(h, 0.0)


def _mlp(x, a0, a1, W1, b1, W2, b2, *, interpret=False):
    return pl.pallas_call(
        _mlp_body,
        grid=(N // R,),
        in_specs=[pl.BlockSpec((R, D), lambda i: (i, 0)),
                  pl.BlockSpec((R, DH), lambda i: (i, 0)),
                  pl.BlockSpec((R, DH), lambda i: (i, 0)),
                  pl.BlockSpec((D, D), lambda i: (0, 0)),
                  pl.BlockSpec((1, D), lambda i: (0, 0)),
                  pl.BlockSpec((D, D), lambda i: (0, 0)),
                  pl.BlockSpec((1, D), lambda i: (0, 0))],
        out_specs=pl.BlockSpec((R, D), lambda i: (i, 0)),
        out_shape=jax.ShapeDtypeStruct((N, D), jnp.float32),
        interpret=interpret,
    )(x, a0, a1, W1, b1.reshape(1, D), W2, b2.reshape(1, D))


def _pool_body(h_ref, batch_ref, wl_ref, bl_ref, o_ref):
    h = h_ref[...]
    b = batch_ref[...]
    gids = lax.broadcasted_iota(jnp.int32, (G, N), 0)
    mask = (gids == b).astype(jnp.float32)
    sums = jnp.dot(mask, h, preferred_element_type=jnp.float32)
    counts = jnp.sum(mask, axis=1, keepdims=True)
    mean = sums / jnp.maximum(counts, 1.0)
    p = jnp.dot(mean, wl_ref[...], preferred_element_type=jnp.float32) + bl_ref[...]
    m = jnp.max(p, axis=1, keepdims=True)
    lse = m + jnp.log(jnp.sum(jnp.exp(p - m), axis=1, keepdims=True))
    o_ref[...] = p - lse


def _pool(h, batch2d, Wl, bl, *, interpret=False):
    return pl.pallas_call(
        _pool_body,
        in_specs=[
            pl.BlockSpec((N, D), lambda: (0, 0)),
            pl.BlockSpec((1, N), lambda: (0, 0)),
            pl.BlockSpec((D, C), lambda: (0, 0)),
            pl.BlockSpec((1, C), lambda: (0, 0)),
        ],
        out_specs=pl.BlockSpec((G, C), lambda: (0, 0)),
        out_shape=jax.ShapeDtypeStruct((G, C), jnp.float32),
        interpret=interpret,
    )(h, batch2d, Wl, bl.reshape(1, C))


def kernel(x, edge_index, batch, W1a, b1a, W2a, b2a, W1b, b1b, W2b, b2b, Wl, bl):
    src = edge_index[0]
    dst = edge_index[1]
    pad = E_PAD - E
    src_p = jnp.concatenate([src, jnp.zeros((pad,), jnp.int32)])
    srcp = jnp.stack([2 * src_p, 2 * src_p + 1]).reshape(NC, NS, JG, 128)
    dstp = jnp.concatenate([dst, jnp.full((pad,), N, jnp.int32)]).reshape(NS, JG, 128)
    zeros = jnp.zeros((NPAD, DH), jnp.float32)

    def agg(h):
        return _agg(h.reshape(2 * N, DH), srcp, dstp, zeros)

    a0, a1 = agg(x)
    h1 = _mlp(x, a0, a1, W1a, b1a, W2a, b2a)
    a0, a1 = agg(h1)
    h2 = _mlp(h1, a0, a1, W1b, b1b, W2b, b2b)
    return _pool(h2, batch.reshape(1, N), Wl, bl)


# speedup vs baseline: 4.1578x; 1.0822x over previous
import functools

import jax
import jax.numpy as jnp
from jax import lax
from jax.experimental import pallas as pl
from jax.experimental.pallas import tpu as pltpu
from jax.experimental.pallas import tpu_sc as plsc

N = 10000
D = 128
E = 320000
G = 64
C = 10

NC, NS = 2, 16
DH = 64
JG = 160
EPT = JG * 128
E_PAD = NS * EPT
NPAD = 10240
ZPT = NPAD // NS


def _agg_body(hv_hbm, src_hbm, dst_hbm, zero_hbm, out0, out1,
              src_v, dst_v, rows_v, acc_sh, gsems, ssems):
    cid = lax.axis_index("c")
    sid = lax.axis_index("s")

    zbase = sid * ZPT
    for g in range(5):
        sl = pl.ds(zbase + g * 128, 128)
        pltpu.sync_copy(zero_hbm.at[sl], rows_v.at[0])
        pltpu.sync_copy(rows_v.at[0], acc_sh.at[sl])
    plsc.subcore_barrier()

    pltpu.sync_copy(src_hbm.at[cid, sid], src_v)
    pltpu.sync_copy(dst_hbm.at[sid], dst_v)

    NB = 4

    def gath(j, b):
        return pltpu.make_async_copy(hv_hbm.at[src_v.at[j]], rows_v.at[b], gsems.at[b])

    def scat(j, b):
        return pltpu.make_async_copy(rows_v.at[b], acc_sh.at[dst_v.at[j]], ssems.at[b])

    for b in range(NB):
        gath(b, b).start()

    def body(i, carry):
        for b in range(NB):
            j = NB * i + b
            gath(j, b).wait()
            pltpu.async_copy(rows_v.at[b], acc_sh.at[dst_v.at[j]], ssems.at[b], add=True)
        for b in range(NB):
            j = NB * i + b
            scat(j, b).wait()

            @pl.when(j + NB < JG)
            def _():
                gath(j + NB, b).start()
        return carry

    lax.fori_loop(0, JG // NB, body, 0)
    plsc.subcore_barrier()

    for g in range(5):
        sl = pl.ds(zbase + g * 128, 128)

        @pl.when(cid == 0)
        def _():
            pltpu.sync_copy(acc_sh.at[sl], rows_v.at[0])
            pltpu.sync_copy(rows_v.at[0], out0.at[sl])

        @pl.when(cid == 1)
        def _():
            pltpu.sync_copy(acc_sh.at[sl], rows_v.at[1])
            pltpu.sync_copy(rows_v.at[1], out1.at[sl])


@functools.cache
def _make_agg():
    return functools.partial(
        pl.kernel,
        out_type=(jax.ShapeDtypeStruct((NPAD, DH), jnp.float32),
                  jax.ShapeDtypeStruct((NPAD, DH), jnp.float32)),
        mesh=plsc.VectorSubcoreMesh(core_axis_name="c", subcore_axis_name="s",
                                    num_cores=NC, num_subcores=NS),
        compiler_params=pltpu.CompilerParams(use_tc_tiling_on_sc=False),
        scratch_types=[
            pltpu.VMEM((JG, 128), jnp.int32),
            pltpu.VMEM((JG, 128), jnp.int32),
            pltpu.VMEM((4, 128, DH), jnp.float32),
            pltpu.VMEM_SHARED((NPAD, DH), jnp.float32),
            pltpu.SemaphoreType.DMA((4,)),
            pltpu.SemaphoreType.DMA((4,)),
        ],
    )(_agg_body)


def _agg(*args):
    return _make_agg()(*args)


R = 1000


def _mlp_body(x_ref, a0_ref, a1_ref, w1_ref, b1_ref, w2_ref, b2_ref, o_ref):
    z = x_ref[...] + jnp.concatenate([a0_ref[...], a1_ref[...]], axis=1)
    t = jnp.dot(z, w1_ref[...], preferred_element_type=jnp.float32) + b1_ref[...]
    t = jnp.maximum(t, 0.0)
    h = jnp.dot(t, w2_ref[...], preferred_element_type=jnp.float32) + b2_ref[...]
    o_ref[...] = jnp.maximum(h, 0.0)


def _mlp(x, a0, a1, W1, b1, W2, b2, *, interpret=False):
    return pl.pallas_call(
        _mlp_body,
        grid=(N // R,),
        in_specs=[pl.BlockSpec((R, D), lambda i: (i, 0)),
                  pl.BlockSpec((R, DH), lambda i: (i, 0)),
                  pl.BlockSpec((R, DH), lambda i: (i, 0)),
                  pl.BlockSpec((D, D), lambda i: (0, 0)),
                  pl.BlockSpec((1, D), lambda i: (0, 0)),
                  pl.BlockSpec((D, D), lambda i: (0, 0)),
                  pl.BlockSpec((1, D), lambda i: (0, 0))],
        out_specs=pl.BlockSpec((R, D), lambda i: (i, 0)),
        out_shape=jax.ShapeDtypeStruct((N, D), jnp.float32),
        interpret=interpret,
    )(x, a0, a1, W1, b1.reshape(1, D), W2, b2.reshape(1, D))


def _pool_body(h_ref, batch_ref, wl_ref, bl_ref, o_ref):
    h = h_ref[...]
    b = batch_ref[...]
    gids = lax.broadcasted_iota(jnp.int32, (G, N), 0)
    mask = (gids == b).astype(jnp.float32)
    sums = jnp.dot(mask, h, preferred_element_type=jnp.float32)
    counts = jnp.sum(mask, axis=1, keepdims=True)
    mean = sums / jnp.maximum(counts, 1.0)
    p = jnp.dot(mean, wl_ref[...], preferred_element_type=jnp.float32) + bl_ref[...]
    m = jnp.max(p, axis=1, keepdims=True)
    lse = m + jnp.log(jnp.sum(jnp.exp(p - m), axis=1, keepdims=True))
    o_ref[...] = p - lse


def _pool(h, batch2d, Wl, bl, *, interpret=False):
    return pl.pallas_call(
        _pool_body,
        in_specs=[
            pl.BlockSpec((N, D), lambda: (0, 0)),
            pl.BlockSpec((1, N), lambda: (0, 0)),
            pl.BlockSpec((D, C), lambda: (0, 0)),
            pl.BlockSpec((1, C), lambda: (0, 0)),
        ],
        out_specs=pl.BlockSpec((G, C), lambda: (0, 0)),
        out_shape=jax.ShapeDtypeStruct((G, C), jnp.float32),
        interpret=interpret,
    )(h, batch2d, Wl, bl.reshape(1, C))


def kernel(x, edge_index, batch, W1a, b1a, W2a, b2a, W1b, b1b, W2b, b2b, Wl, bl):
    src = edge_index[0]
    dst = edge_index[1]
    pad = E_PAD - E
    src_p = jnp.concatenate([src, jnp.zeros((pad,), jnp.int32)])
    srcp = jnp.stack([2 * src_p, 2 * src_p + 1]).reshape(NC, NS, JG, 128)
    dstp = jnp.concatenate([dst, jnp.full((pad,), N, jnp.int32)]).reshape(NS, JG, 128)
    zeros = jnp.zeros((NPAD, DH), jnp.float32)

    def agg(h):
        return _agg(h.reshape(2 * N, DH), srcp, dstp, zeros)

    a0, a1 = agg(x)
    h1 = _mlp(x, a0, a1, W1a, b1a, W2a, b2a)
    a0, a1 = agg(h1)
    h2 = _mlp(h1, a0, a1, W1b, b1b, W2b, b2b)
    return _pool(h2, batch.reshape(1, N), Wl, bl)


# speedup vs baseline: 7.3724x; 1.7731x over previous
import functools

import jax
import jax.numpy as jnp
from jax import lax
from jax.experimental import pallas as pl
from jax.experimental.pallas import tpu as pltpu
from jax.experimental.pallas import tpu_sc as plsc

N = 10000
D = 128
E = 320000
G = 64
C = 10

NC, NS = 2, 16
DH = 64
JG = 160
EPT = JG * 128
E_PAD = NS * EPT
NPAD = 10240
ZPT = NPAD // NS
CH = 16
NB = 4


def _agg_body(hp_hbm, src_hbm, dst_hbm, out0, out1,
              src_cv, dst_cv, rows_v, tab_sh, acc_sh, gsems, ssems, isems):
    cid = lax.axis_index("c")
    sid = lax.axis_index("s")

    zbase = sid * ZPT
    for g in range(5):
        sl = pl.ds(zbase + g * 128, 128)
        pltpu.sync_copy(hp_hbm.at[cid, sl], rows_v.at[0])
        pltpu.sync_copy(rows_v.at[0], tab_sh.at[sl])
        pltpu.sync_copy(rows_v.at[0], acc_sh.at[sl])
    plsc.subcore_barrier()

    def idx_load(i, p):
        pltpu.async_copy(src_hbm.at[sid, pl.ds(i * CH, CH)], src_cv.at[p], isems.at[p])
        pltpu.async_copy(dst_hbm.at[sid, pl.ds(i * CH, CH)], dst_cv.at[p], isems.at[p])

    def idx_wait(i, p):
        pltpu.make_async_copy(src_hbm.at[sid, pl.ds(i * CH, CH)], src_cv.at[p], isems.at[p]).wait()
        pltpu.make_async_copy(dst_hbm.at[sid, pl.ds(i * CH, CH)], dst_cv.at[p], isems.at[p]).wait()

    idx_load(0, 0)

    def gath(p, jj, b):
        return pltpu.make_async_copy(tab_sh.at[src_cv.at[p, jj]], rows_v.at[b], gsems.at[b])

    def scat(p, jj, b):
        return pltpu.make_async_copy(rows_v.at[b], acc_sh.at[dst_cv.at[p, jj]], ssems.at[b])

    NCHUNK = JG // CH

    def body(i, carry):
        p = lax.rem(i, 2)
        idx_wait(i, p)

        for b in range(NB):
            @pl.when(i > 0)
            def _():
                scat(1 - p, CH - NB + b, b).wait()

            gath(p, b, b).start()

        @pl.when(i < NCHUNK - 1)
        def _():
            idx_load(i + 1, 1 - p)

        for q in range(CH // NB):
            for b in range(NB):
                jj = NB * q + b
                gath(p, jj, b).wait()
                pltpu.async_copy(rows_v.at[b], acc_sh.at[dst_cv.at[p, jj]],
                                 ssems.at[b], add=True)
            if q < CH // NB - 1:
                for b in range(NB):
                    jj = NB * q + b
                    scat(p, jj, b).wait()
                    gath(p, jj + NB, b).start()
        return carry

    lax.fori_loop(0, NCHUNK, body, 0)
    for b in range(NB):
        scat(lax.rem(NCHUNK - 1, 2), CH - NB + b, b).wait()
    plsc.subcore_barrier()

    for g in range(5):
        sl = pl.ds(zbase + g * 128, 128)

        @pl.when(cid == 0)
        def _():
            pltpu.sync_copy(acc_sh.at[sl], rows_v.at[0])
            pltpu.sync_copy(rows_v.at[0], out0.at[sl])

        @pl.when(cid == 1)
        def _():
            pltpu.sync_copy(acc_sh.at[sl], rows_v.at[1])
            pltpu.sync_copy(rows_v.at[1], out1.at[sl])


@functools.cache
def _make_agg():
    return functools.partial(
        pl.kernel,
        out_type=(jax.ShapeDtypeStruct((NPAD, DH), jnp.float32),
                  jax.ShapeDtypeStruct((NPAD, DH), jnp.float32)),
        mesh=plsc.VectorSubcoreMesh(core_axis_name="c", subcore_axis_name="s",
                                    num_cores=NC, num_subcores=NS),
        compiler_params=pltpu.CompilerParams(use_tc_tiling_on_sc=False),
        scratch_types=[
            pltpu.VMEM((2, CH, 128), jnp.int32),
            pltpu.VMEM((2, CH, 128), jnp.int32),
            pltpu.VMEM((NB, 128, DH), jnp.float32),
            pltpu.VMEM_SHARED((NPAD, DH), jnp.float32),
            pltpu.VMEM_SHARED((NPAD, DH), jnp.float32),
            pltpu.SemaphoreType.DMA((NB,)),
            pltpu.SemaphoreType.DMA((NB,)),
            pltpu.SemaphoreType.DMA((2,)),
        ],
    )(_agg_body)


def _agg(*args):
    return _make_agg()(*args)


R = 1000


def _mlp_body(a0_ref, a1_ref, w1_ref, b1_ref, w2_ref, b2_ref, o_ref):
    z = jnp.concatenate([a0_ref[...], a1_ref[...]], axis=1)
    t = jnp.dot(z, w1_ref[...], preferred_element_type=jnp.float32) + b1_ref[...]
    t = jnp.maximum(t, 0.0)
    h = jnp.dot(t, w2_ref[...], preferred_element_type=jnp.float32) + b2_ref[...]
    o_ref[...] = jnp.maximum(h, 0.0)


def _mlp(a0, a1, W1, b1, W2, b2, *, interpret=False):
    return pl.pallas_call(
        _mlp_body,
        grid=(N // R,),
        in_specs=[pl.BlockSpec((R, DH), lambda i: (i, 0)),
                  pl.BlockSpec((R, DH), lambda i: (i, 0)),
                  pl.BlockSpec((D, D), lambda i: (0, 0)),
                  pl.BlockSpec((1, D), lambda i: (0, 0)),
                  pl.BlockSpec((D, D), lambda i: (0, 0)),
                  pl.BlockSpec((1, D), lambda i: (0, 0))],
        out_specs=pl.BlockSpec((R, D), lambda i: (i, 0)),
        out_shape=jax.ShapeDtypeStruct((N, D), jnp.float32),
        interpret=interpret,
    )(a0, a1, W1, b1.reshape(1, D), W2, b2.reshape(1, D))


def _pool_body(h_ref, batch_ref, wl_ref, bl_ref, o_ref):
    h = h_ref[...]
    b = batch_ref[...]
    gids = lax.broadcasted_iota(jnp.int32, (G, N), 0)
    mask = (gids == b).astype(jnp.float32)
    sums = jnp.dot(mask, h, preferred_element_type=jnp.float32)
    counts = jnp.sum(mask, axis=1, keepdims=True)
    mean = sums / jnp.maximum(counts, 1.0)
    p = jnp.dot(mean, wl_ref[...], preferred_element_type=jnp.float32) + bl_ref[...]
    m = jnp.max(p, axis=1, keepdims=True)
    lse = m + jnp.log(jnp.sum(jnp.exp(p - m), axis=1, keepdims=True))
    o_ref[...] = p - lse


def _pool(h, batch2d, Wl, bl, *, interpret=False):
    return pl.pallas_call(
        _pool_body,
        in_specs=[
            pl.BlockSpec((N, D), lambda: (0, 0)),
            pl.BlockSpec((1, N), lambda: (0, 0)),
            pl.BlockSpec((D, C), lambda: (0, 0)),
            pl.BlockSpec((1, C), lambda: (0, 0)),
        ],
        out_specs=pl.BlockSpec((G, C), lambda: (0, 0)),
        out_shape=jax.ShapeDtypeStruct((G, C), jnp.float32),
        interpret=interpret,
    )(h, batch2d, Wl, bl.reshape(1, C))


def kernel(x, edge_index, batch, W1a, b1a, W2a, b2a, W1b, b1b, W2b, b2b, Wl, bl):
    src = edge_index[0]
    dst = edge_index[1]
    pad = E_PAD - E
    srcp = jnp.concatenate([src, jnp.zeros((pad,), jnp.int32)]).reshape(NS, JG, 128)
    dstp = jnp.concatenate([dst, jnp.full((pad,), N, jnp.int32)]).reshape(NS, JG, 128)

    def agg(h):
        hp = jnp.zeros((NC, NPAD, DH), jnp.float32)
        hp = hp.at[0, :N].set(h[:, :DH]).at[1, :N].set(h[:, DH:])
        return _agg(hp, srcp, dstp)

    a0, a1 = agg(x)
    h1 = _mlp(a0, a1, W1a, b1a, W2a, b2a)
    a0, a1 = agg(h1)
    h2 = _mlp(a0, a1, W1b, b1b, W2b, b2b)
    return _pool(h2, batch.reshape(1, N), Wl, bl)


# speedup vs baseline: 11.9547x; 1.6215x over previous
import functools

import jax
import jax.numpy as jnp
from jax import lax
from jax.experimental import pallas as pl
from jax.experimental.pallas import tpu as pltpu
from jax.experimental.pallas import tpu_sc as plsc

N = 10000
D = 128
E = 320000
G = 64
C = 10

NC, NS = 2, 16
DH = 64
JG = 160
EPT = JG * 128
E_PAD = NS * EPT
NPAD = 10240
ZPT = NPAD // NS
CH = 16
NB = 4


def _agg_body(hp_hbm, src_hbm, dst_hbm, out0, out1,
              src_cv, dst_cv, rows_v, tab_sh, acc_sh, gsems, ssems, isems):
    cid = lax.axis_index("c")
    sid = lax.axis_index("s")

    zbase = sid * ZPT
    for g in range(5):
        sl = pl.ds(zbase + g * 128, 128)
        pltpu.sync_copy(hp_hbm.at[cid, sl], rows_v.at[0])
        pltpu.sync_copy(rows_v.at[0], tab_sh.at[sl])
        pltpu.sync_copy(rows_v.at[0], acc_sh.at[sl])
    plsc.subcore_barrier()

    def idx_load(i, p):
        pltpu.async_copy(src_hbm.at[sid, pl.ds(i * CH, CH)], src_cv.at[p], isems.at[p])
        pltpu.async_copy(dst_hbm.at[sid, pl.ds(i * CH, CH)], dst_cv.at[p], isems.at[p])

    def idx_wait(i, p):
        pltpu.make_async_copy(src_hbm.at[sid, pl.ds(i * CH, CH)], src_cv.at[p], isems.at[p]).wait()
        pltpu.make_async_copy(dst_hbm.at[sid, pl.ds(i * CH, CH)], dst_cv.at[p], isems.at[p]).wait()

    idx_load(0, 0)

    def gath(p, jj, b):
        return pltpu.make_async_copy(tab_sh.at[src_cv.at[p, jj]], rows_v.at[b], gsems.at[b])

    def scat(p, jj, b):
        return pltpu.make_async_copy(rows_v.at[b], acc_sh.at[dst_cv.at[p, jj]], ssems.at[b])

    NCHUNK = JG // CH

    def body(i, carry):
        p = lax.rem(i, 2)
        idx_wait(i, p)

        for b in range(NB):
            @pl.when(i > 0)
            def _():
                scat(1 - p, CH - NB + b, b).wait()

            gath(p, b, b).start()

        @pl.when(i < NCHUNK - 1)
        def _():
            idx_load(i + 1, 1 - p)

        for q in range(CH // NB):
            for b in range(NB):
                jj = NB * q + b
                gath(p, jj, b).wait()
                pltpu.async_copy(rows_v.at[b], acc_sh.at[dst_cv.at[p, jj]],
                                 ssems.at[b], add=True)
            if q < CH // NB - 1:
                for b in range(NB):
                    jj = NB * q + b
                    scat(p, jj, b).wait()
                    gath(p, jj + NB, b).start()
        return carry

    lax.fori_loop(0, NCHUNK, body, 0)
    for b in range(NB):
        scat(lax.rem(NCHUNK - 1, 2), CH - NB + b, b).wait()
    plsc.subcore_barrier()

    for g in range(5):
        sl = pl.ds(zbase + g * 128, 128)

        @pl.when(cid == 0)
        def _():
            pltpu.sync_copy(acc_sh.at[sl], rows_v.at[0])
            pltpu.sync_copy(rows_v.at[0], out0.at[sl])

        @pl.when(cid == 1)
        def _():
            pltpu.sync_copy(acc_sh.at[sl], rows_v.at[1])
            pltpu.sync_copy(rows_v.at[1], out1.at[sl])


@functools.cache
def _make_agg():
    return functools.partial(
        pl.kernel,
        out_type=(jax.ShapeDtypeStruct((NPAD, DH), jnp.bfloat16),
                  jax.ShapeDtypeStruct((NPAD, DH), jnp.bfloat16)),
        mesh=plsc.VectorSubcoreMesh(core_axis_name="c", subcore_axis_name="s",
                                    num_cores=NC, num_subcores=NS),
        compiler_params=pltpu.CompilerParams(use_tc_tiling_on_sc=False),
        scratch_types=[
            pltpu.VMEM((2, CH, 128), jnp.int32),
            pltpu.VMEM((2, CH, 128), jnp.int32),
            pltpu.VMEM((NB, 128, DH), jnp.bfloat16),
            pltpu.VMEM_SHARED((NPAD, DH), jnp.bfloat16),
            pltpu.VMEM_SHARED((NPAD, DH), jnp.bfloat16),
            pltpu.SemaphoreType.DMA((NB,)),
            pltpu.SemaphoreType.DMA((NB,)),
            pltpu.SemaphoreType.DMA((2,)),
        ],
    )(_agg_body)


def _agg(*args):
    return _make_agg()(*args)


R = 1000


def _mlp_body(a0_ref, a1_ref, w1_ref, b1_ref, w2_ref, b2_ref, o_ref):
    z = jnp.concatenate([a0_ref[...], a1_ref[...]], axis=1).astype(jnp.float32)
    t = jnp.dot(z, w1_ref[...], preferred_element_type=jnp.float32) + b1_ref[...]
    t = jnp.maximum(t, 0.0)
    h = jnp.dot(t, w2_ref[...], preferred_element_type=jnp.float32) + b2_ref[...]
    o_ref[...] = jnp.maximum(h, 0.0)


def _mlp(a0, a1, W1, b1, W2, b2, *, interpret=False):
    return pl.pallas_call(
        _mlp_body,
        grid=(N // R,),
        in_specs=[pl.BlockSpec((R, DH), lambda i: (i, 0)),
                  pl.BlockSpec((R, DH), lambda i: (i, 0)),
                  pl.BlockSpec((D, D), lambda i: (0, 0)),
                  pl.BlockSpec((1, D), lambda i: (0, 0)),
                  pl.BlockSpec((D, D), lambda i: (0, 0)),
                  pl.BlockSpec((1, D), lambda i: (0, 0))],
        out_specs=pl.BlockSpec((R, D), lambda i: (i, 0)),
        out_shape=jax.ShapeDtypeStruct((N, D), jnp.float32),
        interpret=interpret,
    )(a0, a1, W1, b1.reshape(1, D), W2, b2.reshape(1, D))


def _pool_body(h_ref, batch_ref, wl_ref, bl_ref, o_ref):
    h = h_ref[...]
    b = batch_ref[...]
    gids = lax.broadcasted_iota(jnp.int32, (G, N), 0)
    mask = (gids == b).astype(jnp.float32)
    sums = jnp.dot(mask, h, preferred_element_type=jnp.float32)
    counts = jnp.sum(mask, axis=1, keepdims=True)
    mean = sums / jnp.maximum(counts, 1.0)
    p = jnp.dot(mean, wl_ref[...], preferred_element_type=jnp.float32) + bl_ref[...]
    m = jnp.max(p, axis=1, keepdims=True)
    lse = m + jnp.log(jnp.sum(jnp.exp(p - m), axis=1, keepdims=True))
    o_ref[...] = p - lse


def _pool(h, batch2d, Wl, bl, *, interpret=False):
    return pl.pallas_call(
        _pool_body,
        in_specs=[
            pl.BlockSpec((N, D), lambda: (0, 0)),
            pl.BlockSpec((1, N), lambda: (0, 0)),
            pl.BlockSpec((D, C), lambda: (0, 0)),
            pl.BlockSpec((1, C), lambda: (0, 0)),
        ],
        out_specs=pl.BlockSpec((G, C), lambda: (0, 0)),
        out_shape=jax.ShapeDtypeStruct((G, C), jnp.float32),
        interpret=interpret,
    )(h, batch2d, Wl, bl.reshape(1, C))


def kernel(x, edge_index, batch, W1a, b1a, W2a, b2a, W1b, b1b, W2b, b2b, Wl, bl):
    src = edge_index[0]
    dst = edge_index[1]
    pad = E_PAD - E
    srcp = jnp.concatenate([src, jnp.zeros((pad,), jnp.int32)]).reshape(NS, JG, 128)
    dstp = jnp.concatenate([dst, jnp.full((pad,), N, jnp.int32)]).reshape(NS, JG, 128)

    def agg(h):
        hb = h.astype(jnp.bfloat16)
        hp = jnp.zeros((NC, NPAD, DH), jnp.bfloat16)
        hp = hp.at[0, :N].set(hb[:, :DH]).at[1, :N].set(hb[:, DH:])
        return _agg(hp, srcp, dstp)

    a0, a1 = agg(x)
    h1 = _mlp(a0, a1, W1a, b1a, W2a, b2a)
    a0, a1 = agg(h1)
    h2 = _mlp(a0, a1, W1b, b1b, W2b, b2b)
    return _pool(h2, batch.reshape(1, N), Wl, bl)


# speedup vs baseline: 12.5276x; 1.0479x over previous
import functools

import jax
import jax.numpy as jnp
from jax import lax
from jax.experimental import pallas as pl
from jax.experimental.pallas import tpu as pltpu
from jax.experimental.pallas import tpu_sc as plsc

N = 10000
D = 128
E = 320000
G = 64
C = 10

NC, NS = 2, 16
DH = 64
JG = 160
EPT = JG * 128
E_PAD = NS * EPT
NPAD = 10240
ZPT = NPAD // NS
CH = 32
NB = 8


def _agg_body(hp_hbm, src_hbm, dst_hbm, out0, out1,
              src_cv, dst_cv, rows_v, tab_sh, acc_sh, gsems, ssems, isems):
    cid = lax.axis_index("c")
    sid = lax.axis_index("s")

    zbase = sid * ZPT
    for g in range(5):
        sl = pl.ds(zbase + g * 128, 128)
        pltpu.sync_copy(hp_hbm.at[cid, sl], rows_v.at[0])
        pltpu.sync_copy(rows_v.at[0], tab_sh.at[sl])
        pltpu.sync_copy(rows_v.at[0], acc_sh.at[sl])
    plsc.subcore_barrier()

    def idx_load(i, p):
        pltpu.async_copy(src_hbm.at[sid, pl.ds(i * CH, CH)], src_cv.at[p], isems.at[p])
        pltpu.async_copy(dst_hbm.at[sid, pl.ds(i * CH, CH)], dst_cv.at[p], isems.at[p])

    def idx_wait(i, p):
        pltpu.make_async_copy(src_hbm.at[sid, pl.ds(i * CH, CH)], src_cv.at[p], isems.at[p]).wait()
        pltpu.make_async_copy(dst_hbm.at[sid, pl.ds(i * CH, CH)], dst_cv.at[p], isems.at[p]).wait()

    idx_load(0, 0)

    def gath(p, jj, b):
        return pltpu.make_async_copy(tab_sh.at[src_cv.at[p, jj]], rows_v.at[b], gsems.at[b])

    def scat(p, jj, b):
        return pltpu.make_async_copy(rows_v.at[b], acc_sh.at[dst_cv.at[p, jj]], ssems.at[b])

    NCHUNK = JG // CH

    def body(i, carry):
        p = lax.rem(i, 2)
        idx_wait(i, p)

        for b in range(NB):
            @pl.when(i > 0)
            def _():
                scat(1 - p, CH - NB + b, b).wait()

            gath(p, b, b).start()

        @pl.when(i < NCHUNK - 1)
        def _():
            idx_load(i + 1, 1 - p)

        for q in range(CH // NB):
            for b in range(NB):
                jj = NB * q + b
                gath(p, jj, b).wait()
                pltpu.async_copy(rows_v.at[b], acc_sh.at[dst_cv.at[p, jj]],
                                 ssems.at[b], add=True)
            if q < CH // NB - 1:
                for b in range(NB):
                    jj = NB * q + b
                    scat(p, jj, b).wait()
                    gath(p, jj + NB, b).start()
        return carry

    lax.fori_loop(0, NCHUNK, body, 0)
    for b in range(NB):
        scat(lax.rem(NCHUNK - 1, 2), CH - NB + b, b).wait()
    plsc.subcore_barrier()

    for g in range(5):
        sl = pl.ds(zbase + g * 128, 128)

        @pl.when(cid == 0)
        def _():
            pltpu.sync_copy(acc_sh.at[sl], rows_v.at[0])
            pltpu.sync_copy(rows_v.at[0], out0.at[sl])

        @pl.when(cid == 1)
        def _():
            pltpu.sync_copy(acc_sh.at[sl], rows_v.at[1])
            pltpu.sync_copy(rows_v.at[1], out1.at[sl])


@functools.cache
def _make_agg():
    return functools.partial(
        pl.kernel,
        out_type=(jax.ShapeDtypeStruct((NPAD, DH), jnp.bfloat16),
                  jax.ShapeDtypeStruct((NPAD, DH), jnp.bfloat16)),
        mesh=plsc.VectorSubcoreMesh(core_axis_name="c", subcore_axis_name="s",
                                    num_cores=NC, num_subcores=NS),
        compiler_params=pltpu.CompilerParams(use_tc_tiling_on_sc=False),
        scratch_types=[
            pltpu.VMEM((2, CH, 128), jnp.int32),
            pltpu.VMEM((2, CH, 128), jnp.int32),
            pltpu.VMEM((NB, 128, DH), jnp.bfloat16),
            pltpu.VMEM_SHARED((NPAD, DH), jnp.bfloat16),
            pltpu.VMEM_SHARED((NPAD, DH), jnp.bfloat16),
            pltpu.SemaphoreType.DMA((NB,)),
            pltpu.SemaphoreType.DMA((NB,)),
            pltpu.SemaphoreType.DMA((2,)),
        ],
    )(_agg_body)


def _agg(*args):
    return _make_agg()(*args)


R = 1000


def _mlp_body(a0_ref, a1_ref, w1_ref, b1_ref, w2_ref, b2_ref, o_ref):
    z = jnp.concatenate([a0_ref[...], a1_ref[...]], axis=1).astype(jnp.float32)
    t = jnp.dot(z, w1_ref[...], preferred_element_type=jnp.float32) + b1_ref[...]
    t = jnp.maximum(t, 0.0)
    h = jnp.dot(t, w2_ref[...], preferred_element_type=jnp.float32) + b2_ref[...]
    o_ref[...] = jnp.maximum(h, 0.0)


def _mlp(a0, a1, W1, b1, W2, b2, *, interpret=False):
    return pl.pallas_call(
        _mlp_body,
        grid=(N // R,),
        in_specs=[pl.BlockSpec((R, DH), lambda i: (i, 0)),
                  pl.BlockSpec((R, DH), lambda i: (i, 0)),
                  pl.BlockSpec((D, D), lambda i: (0, 0)),
                  pl.BlockSpec((1, D), lambda i: (0, 0)),
                  pl.BlockSpec((D, D), lambda i: (0, 0)),
                  pl.BlockSpec((1, D), lambda i: (0, 0))],
        out_specs=pl.BlockSpec((R, D), lambda i: (i, 0)),
        out_shape=jax.ShapeDtypeStruct((N, D), jnp.float32),
        interpret=interpret,
    )(a0, a1, W1, b1.reshape(1, D), W2, b2.reshape(1, D))


def _pool_body(h_ref, batch_ref, wl_ref, bl_ref, o_ref):
    h = h_ref[...]
    b = batch_ref[...]
    gids = lax.broadcasted_iota(jnp.int32, (G, N), 0)
    mask = (gids == b).astype(jnp.float32)
    sums = jnp.dot(mask, h, preferred_element_type=jnp.float32)
    counts = jnp.sum(mask, axis=1, keepdims=True)
    mean = sums / jnp.maximum(counts, 1.0)
    p = jnp.dot(mean, wl_ref[...], preferred_element_type=jnp.float32) + bl_ref[...]
    m = jnp.max(p, axis=1, keepdims=True)
    lse = m + jnp.log(jnp.sum(jnp.exp(p - m), axis=1, keepdims=True))
    o_ref[...] = p - lse


def _pool(h, batch2d, Wl, bl, *, interpret=False):
    return pl.pallas_call(
        _pool_body,
        in_specs=[
            pl.BlockSpec((N, D), lambda: (0, 0)),
            pl.BlockSpec((1, N), lambda: (0, 0)),
            pl.BlockSpec((D, C), lambda: (0, 0)),
            pl.BlockSpec((1, C), lambda: (0, 0)),
        ],
        out_specs=pl.BlockSpec((G, C), lambda: (0, 0)),
        out_shape=jax.ShapeDtypeStruct((G, C), jnp.float32),
        interpret=interpret,
    )(h, batch2d, Wl, bl.reshape(1, C))


def kernel(x, edge_index, batch, W1a, b1a, W2a, b2a, W1b, b1b, W2b, b2b, Wl, bl):
    src = edge_index[0]
    dst = edge_index[1]
    pad = E_PAD - E
    srcp = jnp.concatenate([src, jnp.zeros((pad,), jnp.int32)]).reshape(NS, JG, 128)
    dstp = jnp.concatenate([dst, jnp.full((pad,), N, jnp.int32)]).reshape(NS, JG, 128)

    def agg(h):
        hb = h.astype(jnp.bfloat16)
        hp = jnp.zeros((NC, NPAD, DH), jnp.bfloat16)
        hp = hp.at[0, :N].set(hb[:, :DH]).at[1, :N].set(hb[:, DH:])
        return _agg(hp, srcp, dstp)

    a0, a1 = agg(x)
    h1 = _mlp(a0, a1, W1a, b1a, W2a, b2a)
    a0, a1 = agg(h1)
    h2 = _mlp(a0, a1, W1b, b1b, W2b, b2b)
    return _pool(h2, batch.reshape(1, N), Wl, bl)


# speedup vs baseline: 13.2761x; 1.0597x over previous
import functools

import jax
import jax.numpy as jnp
from jax import lax
from jax.experimental import pallas as pl
from jax.experimental.pallas import tpu as pltpu
from jax.experimental.pallas import tpu_sc as plsc

N = 10000
D = 128
E = 320000
G = 64
C = 10

NC, NS = 2, 16
DH = 64
JG = 160
EPT = JG * 128
E_PAD = NS * EPT
NPAD = 10240
ZPT = NPAD // NS
CH = 32
NB = 8


def _agg_body(hp_hbm, src_hbm, dst_hbm, out0, out1,
              src_cv, dst_cv, rows_v, tab_sh, acc_sh, gsems, ssems, isems):
    cid = lax.axis_index("c")
    sid = lax.axis_index("s")

    zbase = sid * ZPT
    for g in range(5):
        sl = pl.ds(zbase + g * 128, 128)
        pltpu.async_copy(hp_hbm.at[cid, sl], rows_v.at[g], gsems.at[g])
    for g in range(5):
        sl = pl.ds(zbase + g * 128, 128)
        pltpu.make_async_copy(hp_hbm.at[cid, sl], rows_v.at[g], gsems.at[g]).wait()
        pltpu.async_copy(rows_v.at[g], tab_sh.at[sl], ssems.at[g])
        pltpu.async_copy(rows_v.at[g], acc_sh.at[sl], ssems.at[g + 1])
    for g in range(5):
        sl = pl.ds(zbase + g * 128, 128)
        pltpu.make_async_copy(rows_v.at[g], tab_sh.at[sl], ssems.at[g]).wait()
        pltpu.make_async_copy(rows_v.at[g], acc_sh.at[sl], ssems.at[g + 1]).wait()
    plsc.subcore_barrier()

    def idx_load(i, p):
        pltpu.async_copy(src_hbm.at[sid, pl.ds(i * CH, CH)], src_cv.at[p], isems.at[p])
        pltpu.async_copy(dst_hbm.at[sid, pl.ds(i * CH, CH)], dst_cv.at[p], isems.at[p])

    def idx_wait(i, p):
        pltpu.make_async_copy(src_hbm.at[sid, pl.ds(i * CH, CH)], src_cv.at[p], isems.at[p]).wait()
        pltpu.make_async_copy(dst_hbm.at[sid, pl.ds(i * CH, CH)], dst_cv.at[p], isems.at[p]).wait()

    idx_load(0, 0)

    def gath(p, jj, b):
        return pltpu.make_async_copy(tab_sh.at[src_cv.at[p, jj]], rows_v.at[b], gsems.at[b])

    def scat(p, jj, b):
        return pltpu.make_async_copy(rows_v.at[b], acc_sh.at[dst_cv.at[p, jj]], ssems.at[b])

    NCHUNK = JG // CH

    def body(i, carry):
        p = lax.rem(i, 2)
        idx_wait(i, p)

        for b in range(NB):
            @pl.when(i > 0)
            def _():
                scat(1 - p, CH - NB + b, b).wait()

            gath(p, b, b).start()

        @pl.when(i < NCHUNK - 1)
        def _():
            idx_load(i + 1, 1 - p)

        for q in range(CH // NB):
            for b in range(NB):
                jj = NB * q + b
                gath(p, jj, b).wait()
                pltpu.async_copy(rows_v.at[b], acc_sh.at[dst_cv.at[p, jj]],
                                 ssems.at[b], add=True)
            if q < CH // NB - 1:
                for b in range(NB):
                    jj = NB * q + b
                    scat(p, jj, b).wait()
                    gath(p, jj + NB, b).start()
        return carry

    lax.fori_loop(0, NCHUNK, body, 0)
    for b in range(NB):
        scat(lax.rem(NCHUNK - 1, 2), CH - NB + b, b).wait()
    plsc.subcore_barrier()

    for g in range(5):
        sl = pl.ds(zbase + g * 128, 128)
        pltpu.async_copy(acc_sh.at[sl], rows_v.at[g], gsems.at[g])
    for g in range(5):
        sl = pl.ds(zbase + g * 128, 128)
        pltpu.make_async_copy(acc_sh.at[sl], rows_v.at[g], gsems.at[g]).wait()

        @pl.when(cid == 0)
        def _():
            pltpu.async_copy(rows_v.at[g], out0.at[sl], ssems.at[g])

        @pl.when(cid == 1)
        def _():
            pltpu.async_copy(rows_v.at[g], out1.at[sl], ssems.at[g])
    for g in range(5):
        sl = pl.ds(zbase + g * 128, 128)

        @pl.when(cid == 0)
        def _():
            pltpu.make_async_copy(rows_v.at[g], out0.at[sl], ssems.at[g]).wait()

        @pl.when(cid == 1)
        def _():
            pltpu.make_async_copy(rows_v.at[g], out1.at[sl], ssems.at[g]).wait()


@functools.cache
def _make_agg():
    return functools.partial(
        pl.kernel,
        out_type=(jax.ShapeDtypeStruct((NPAD, DH), jnp.bfloat16),
                  jax.ShapeDtypeStruct((NPAD, DH), jnp.bfloat16)),
        mesh=plsc.VectorSubcoreMesh(core_axis_name="c", subcore_axis_name="s",
                                    num_cores=NC, num_subcores=NS),
        compiler_params=pltpu.CompilerParams(use_tc_tiling_on_sc=False),
        scratch_types=[
            pltpu.VMEM((2, CH, 128), jnp.int32),
            pltpu.VMEM((2, CH, 128), jnp.int32),
            pltpu.VMEM((NB, 128, DH), jnp.bfloat16),
            pltpu.VMEM_SHARED((NPAD, DH), jnp.bfloat16),
            pltpu.VMEM_SHARED((NPAD, DH), jnp.bfloat16),
            pltpu.SemaphoreType.DMA((NB,)),
            pltpu.SemaphoreType.DMA((NB,)),
            pltpu.SemaphoreType.DMA((2,)),
        ],
    )(_agg_body)


def _agg(*args):
    return _make_agg()(*args)


R = 2000


def _mlp_body(a0_ref, a1_ref, w1_ref, b1_ref, w2_ref, b2_ref, o_ref, hp_ref):
    z = jnp.concatenate([a0_ref[...], a1_ref[...]], axis=1).astype(jnp.float32)
    t = jnp.dot(z, w1_ref[...], preferred_element_type=jnp.float32) + b1_ref[...]
    t = jnp.maximum(t, 0.0)
    h = jnp.dot(t, w2_ref[...], preferred_element_type=jnp.float32) + b2_ref[...]
    h = jnp.maximum(h, 0.0)
    o_ref[...] = h
    hb = h.astype(jnp.bfloat16)
    hp_ref[...] = jnp.stack([hb[:, :DH], hb[:, DH:]])


def _mlp(a0, a1, W1, b1, W2, b2, *, interpret=False):
    return pl.pallas_call(
        _mlp_body,
        grid=(N // R,),
        in_specs=[pl.BlockSpec((R, DH), lambda i: (i, 0)),
                  pl.BlockSpec((R, DH), lambda i: (i, 0)),
                  pl.BlockSpec((D, D), lambda i: (0, 0)),
                  pl.BlockSpec((1, D), lambda i: (0, 0)),
                  pl.BlockSpec((D, D), lambda i: (0, 0)),
                  pl.BlockSpec((1, D), lambda i: (0, 0))],
        out_specs=[pl.BlockSpec((R, D), lambda i: (i, 0)),
                   pl.BlockSpec((NC, R, DH), lambda i: (0, i, 0))],
        out_shape=[jax.ShapeDtypeStruct((N, D), jnp.float32),
                   jax.ShapeDtypeStruct((NC, NPAD, DH), jnp.bfloat16)],
        interpret=interpret,
    )(a0, a1, W1, b1.reshape(1, D), W2, b2.reshape(1, D))


def _pool_body(h_ref, batch_ref, wl_ref, bl_ref, o_ref):
    h = h_ref[...]
    b = batch_ref[...]
    gids = lax.broadcasted_iota(jnp.int32, (G, N), 0)
    mask = (gids == b).astype(jnp.float32)
    sums = jnp.dot(mask, h, preferred_element_type=jnp.float32)
    counts = jnp.sum(mask, axis=1, keepdims=True)
    mean = sums / jnp.maximum(counts, 1.0)
    p = jnp.dot(mean, wl_ref[...], preferred_element_type=jnp.float32) + bl_ref[...]
    m = jnp.max(p, axis=1, keepdims=True)
    lse = m + jnp.log(jnp.sum(jnp.exp(p - m), axis=1, keepdims=True))
    o_ref[...] = p - lse


def _pool(h, batch2d, Wl, bl, *, interpret=False):
    return pl.pallas_call(
        _pool_body,
        in_specs=[
            pl.BlockSpec((N, D), lambda: (0, 0)),
            pl.BlockSpec((1, N), lambda: (0, 0)),
            pl.BlockSpec((D, C), lambda: (0, 0)),
            pl.BlockSpec((1, C), lambda: (0, 0)),
        ],
        out_specs=pl.BlockSpec((G, C), lambda: (0, 0)),
        out_shape=jax.ShapeDtypeStruct((G, C), jnp.float32),
        interpret=interpret,
    )(h, batch2d, Wl, bl.reshape(1, C))


def kernel(x, edge_index, batch, W1a, b1a, W2a, b2a, W1b, b1b, W2b, b2b, Wl, bl):
    src = edge_index[0]
    dst = edge_index[1]
    pad = E_PAD - E
    srcp = jnp.concatenate([src, jnp.zeros((pad,), jnp.int32)]).reshape(NS, JG, 128)
    dstp = jnp.concatenate([dst, jnp.full((pad,), N, jnp.int32)]).reshape(NS, JG, 128)

    xb = x.astype(jnp.bfloat16)
    xp = jnp.zeros((NC, NPAD, DH), jnp.bfloat16)
    xp = xp.at[0, :N].set(xb[:, :DH]).at[1, :N].set(xb[:, DH:])

    a0, a1 = _agg(xp, srcp, dstp)
    h1, hp1 = _mlp(a0, a1, W1a, b1a, W2a, b2a)
    a0, a1 = _agg(hp1, srcp, dstp)
    h2, _ = _mlp(a0, a1, W1b, b1b, W2b, b2b)
    return _pool(h2, batch.reshape(1, N), Wl, bl)


# speedup vs baseline: 13.5823x; 1.0231x over previous
import functools

import jax
import jax.numpy as jnp
from jax import lax
from jax.experimental import pallas as pl
from jax.experimental.pallas import tpu as pltpu
from jax.experimental.pallas import tpu_sc as plsc

N = 10000
D = 128
E = 320000
G = 64
C = 10

NC, NS = 2, 16
DH = 64
JG = 160
EPT = JG * 128
E_PAD = NS * EPT
NPAD = 10240
ZPT = NPAD // NS
CH = 32
NB = 8


def _agg_body(hp_hbm, src_hbm, dst_hbm, out0, out1,
              src_cv, dst_cv, rows_v, tab_sh, acc_sh, gsems, ssems, isems):
    cid = lax.axis_index("c")
    sid = lax.axis_index("s")

    zbase = sid * ZPT
    for g in range(5):
        sl = pl.ds(zbase + g * 128, 128)
        pltpu.async_copy(hp_hbm.at[cid, sl], rows_v.at[g], gsems.at[g])
    for g in range(5):
        sl = pl.ds(zbase + g * 128, 128)
        pltpu.make_async_copy(hp_hbm.at[cid, sl], rows_v.at[g], gsems.at[g]).wait()
        pltpu.async_copy(rows_v.at[g], tab_sh.at[sl], ssems.at[g])
        pltpu.async_copy(rows_v.at[g], acc_sh.at[sl], ssems.at[g + 1])
    for g in range(5):
        sl = pl.ds(zbase + g * 128, 128)
        pltpu.make_async_copy(rows_v.at[g], tab_sh.at[sl], ssems.at[g]).wait()
        pltpu.make_async_copy(rows_v.at[g], acc_sh.at[sl], ssems.at[g + 1]).wait()
    plsc.subcore_barrier()

    def idx_load(i, p):
        pltpu.async_copy(src_hbm.at[sid, pl.ds(i * CH, CH)], src_cv.at[p], isems.at[p])
        pltpu.async_copy(dst_hbm.at[sid, pl.ds(i * CH, CH)], dst_cv.at[p], isems.at[p])

    def idx_wait(i, p):
        pltpu.make_async_copy(src_hbm.at[sid, pl.ds(i * CH, CH)], src_cv.at[p], isems.at[p]).wait()
        pltpu.make_async_copy(dst_hbm.at[sid, pl.ds(i * CH, CH)], dst_cv.at[p], isems.at[p]).wait()

    idx_load(0, 0)

    def gath(p, jj, b):
        return pltpu.make_async_copy(tab_sh.at[src_cv.at[p, jj]], rows_v.at[b], gsems.at[b])

    def scat(p, jj, b):
        return pltpu.make_async_copy(rows_v.at[b], acc_sh.at[dst_cv.at[p, jj]], ssems.at[b])

    NCHUNK = JG // CH

    def body(i, carry):
        p = lax.rem(i, 2)
        idx_wait(i, p)

        for b in range(NB):
            @pl.when(i > 0)
            def _():
                scat(1 - p, CH - NB + b, b).wait()

            gath(p, b, b).start()

        @pl.when(i < NCHUNK - 1)
        def _():
            idx_load(i + 1, 1 - p)

        for q in range(CH // NB):
            for b in range(NB):
                jj = NB * q + b
                gath(p, jj, b).wait()
                pltpu.async_copy(rows_v.at[b], acc_sh.at[dst_cv.at[p, jj]],
                                 ssems.at[b], add=True)
            if q < CH // NB - 1:
                for b in range(NB):
                    jj = NB * q + b
                    scat(p, jj, b).wait()
                    gath(p, jj + NB, b).start()
        return carry

    lax.fori_loop(0, NCHUNK, body, 0)
    for b in range(NB):
        scat(lax.rem(NCHUNK - 1, 2), CH - NB + b, b).wait()
    plsc.subcore_barrier()

    for g in range(5):
        sl = pl.ds(zbase + g * 128, 128)
        pltpu.async_copy(acc_sh.at[sl], rows_v.at[g], gsems.at[g])
    for g in range(5):
        sl = pl.ds(zbase + g * 128, 128)
        pltpu.make_async_copy(acc_sh.at[sl], rows_v.at[g], gsems.at[g]).wait()

        @pl.when(cid == 0)
        def _():
            pltpu.async_copy(rows_v.at[g], out0.at[sl], ssems.at[g])

        @pl.when(cid == 1)
        def _():
            pltpu.async_copy(rows_v.at[g], out1.at[sl], ssems.at[g])
    for g in range(5):
        sl = pl.ds(zbase + g * 128, 128)

        @pl.when(cid == 0)
        def _():
            pltpu.make_async_copy(rows_v.at[g], out0.at[sl], ssems.at[g]).wait()

        @pl.when(cid == 1)
        def _():
            pltpu.make_async_copy(rows_v.at[g], out1.at[sl], ssems.at[g]).wait()


@functools.cache
def _make_agg():
    return functools.partial(
        pl.kernel,
        out_type=(jax.ShapeDtypeStruct((NPAD, DH), jnp.bfloat16),
                  jax.ShapeDtypeStruct((NPAD, DH), jnp.bfloat16)),
        mesh=plsc.VectorSubcoreMesh(core_axis_name="c", subcore_axis_name="s",
                                    num_cores=NC, num_subcores=NS),
        compiler_params=pltpu.CompilerParams(use_tc_tiling_on_sc=False),
        scratch_types=[
            pltpu.VMEM((2, CH, 128), jnp.int32),
            pltpu.VMEM((2, CH, 128), jnp.int32),
            pltpu.VMEM((NB, 128, DH), jnp.bfloat16),
            pltpu.VMEM_SHARED((NPAD, DH), jnp.bfloat16),
            pltpu.VMEM_SHARED((NPAD, DH), jnp.bfloat16),
            pltpu.SemaphoreType.DMA((NB,)),
            pltpu.SemaphoreType.DMA((NB,)),
            pltpu.SemaphoreType.DMA((2,)),
        ],
    )(_agg_body)


def _agg(*args):
    return _make_agg()(*args)


R = 2000
NBLK = N // R


def _mlp_mid_body(a0_ref, a1_ref, w1_ref, b1_ref, w2_ref, b2_ref, hp_ref):
    z = jnp.concatenate([a0_ref[...], a1_ref[...]], axis=1).astype(jnp.float32)
    t = jnp.dot(z, w1_ref[...], preferred_element_type=jnp.float32) + b1_ref[...]
    t = jnp.maximum(t, 0.0)
    h = jnp.dot(t, w2_ref[...], preferred_element_type=jnp.float32) + b2_ref[...]
    hb = jnp.maximum(h, 0.0).astype(jnp.bfloat16)
    hp_ref[...] = jnp.stack([hb[:, :DH], hb[:, DH:]])


_W_SPECS = [pl.BlockSpec((D, D), lambda i: (0, 0)),
            pl.BlockSpec((1, D), lambda i: (0, 0)),
            pl.BlockSpec((D, D), lambda i: (0, 0)),
            pl.BlockSpec((1, D), lambda i: (0, 0))]


def _mlp_mid(a0, a1, W1, b1, W2, b2, *, interpret=False):
    return pl.pallas_call(
        _mlp_mid_body,
        grid=(NBLK,),
        in_specs=[pl.BlockSpec((R, DH), lambda i: (i, 0)),
                  pl.BlockSpec((R, DH), lambda i: (i, 0))] + _W_SPECS,
        out_specs=pl.BlockSpec((NC, R, DH), lambda i: (0, i, 0)),
        out_shape=jax.ShapeDtypeStruct((NC, NPAD, DH), jnp.bfloat16),
        interpret=interpret,
    )(a0, a1, W1, b1.reshape(1, D), W2, b2.reshape(1, D))


def _mlp_pool_body(a0_ref, a1_ref, w1_ref, b1_ref, w2_ref, b2_ref,
                   batch_ref, wl_ref, bl_ref, o_ref, sums_ref, cnt_ref):
    i = pl.program_id(0)

    @pl.when(i == 0)
    def _():
        sums_ref[...] = jnp.zeros_like(sums_ref)
        cnt_ref[...] = jnp.zeros_like(cnt_ref)

    z = jnp.concatenate([a0_ref[...], a1_ref[...]], axis=1).astype(jnp.float32)
    t = jnp.dot(z, w1_ref[...], preferred_element_type=jnp.float32) + b1_ref[...]
    t = jnp.maximum(t, 0.0)
    h = jnp.dot(t, w2_ref[...], preferred_element_type=jnp.float32) + b2_ref[...]
    h = jnp.maximum(h, 0.0)
    gids = lax.broadcasted_iota(jnp.int32, (G, R), 0)
    mask = (gids == batch_ref[0]).astype(jnp.float32)
    sums_ref[...] += jnp.dot(mask, h, preferred_element_type=jnp.float32)
    cnt_ref[...] += jnp.sum(mask, axis=1, keepdims=True)

    @pl.when(i == NBLK - 1)
    def _():
        mean = sums_ref[...] / jnp.maximum(cnt_ref[...], 1.0)
        p = jnp.dot(mean, wl_ref[...], preferred_element_type=jnp.float32) + bl_ref[...]
        m = jnp.max(p, axis=1, keepdims=True)
        lse = m + jnp.log(jnp.sum(jnp.exp(p - m), axis=1, keepdims=True))
        o_ref[...] = p - lse


def _mlp_pool(a0, a1, W1, b1, W2, b2, batch3d, Wl, bl, *, interpret=False):
    return pl.pallas_call(
        _mlp_pool_body,
        grid=(NBLK,),
        in_specs=[pl.BlockSpec((R, DH), lambda i: (i, 0)),
                  pl.BlockSpec((R, DH), lambda i: (i, 0))] + _W_SPECS + [
                  pl.BlockSpec((1, 1, R), lambda i: (i, 0, 0)),
                  pl.BlockSpec((D, C), lambda i: (0, 0)),
                  pl.BlockSpec((1, C), lambda i: (0, 0))],
        out_specs=pl.BlockSpec((G, C), lambda i: (0, 0)),
        out_shape=jax.ShapeDtypeStruct((G, C), jnp.float32),
        scratch_shapes=[pltpu.VMEM((G, D), jnp.float32),
                        pltpu.VMEM((G, 1), jnp.float32)],
        interpret=interpret,
    )(a0, a1, W1, b1.reshape(1, D), W2, b2.reshape(1, D),
      batch3d, Wl, bl.reshape(1, C))


def kernel(x, edge_index, batch, W1a, b1a, W2a, b2a, W1b, b1b, W2b, b2b, Wl, bl):
    src = edge_index[0]
    dst = edge_index[1]
    pad = E_PAD - E
    srcp = jnp.concatenate([src, jnp.zeros((pad,), jnp.int32)]).reshape(NS, JG, 128)
    dstp = jnp.concatenate([dst, jnp.full((pad,), N, jnp.int32)]).reshape(NS, JG, 128)

    xb = x.astype(jnp.bfloat16)
    xp = jnp.zeros((NC, NPAD, DH), jnp.bfloat16)
    xp = xp.at[0, :N].set(xb[:, :DH]).at[1, :N].set(xb[:, DH:])

    a0, a1 = _agg(xp, srcp, dstp)
    hp1 = _mlp_mid(a0, a1, W1a, b1a, W2a, b2a)
    a0, a1 = _agg(hp1, srcp, dstp)
    return _mlp_pool(a0, a1, W1b, b1b, W2b, b2b, batch.reshape(NBLK, 1, R), Wl, bl)
